# baseline (speedup 1.0000x reference)
import functools

import jax
import jax.numpy as jnp
from jax import lax
from jax.experimental import pallas as pl
from jax.experimental.pallas import tpu as pltpu
from jax.experimental.pallas import tpu_sc as plsc

N = 10000
E = 320000
D = 128
NPAD = 10240
NC = 2
NS = 16
NW = NC * NS

PBB = 128
PNB = 79
EPWP = PNB * PBB
EPAD = NW * EPWP
ACC = 10112
ZPT = ACC // NS

BB = 80
NBATCH = (E // NW) // BB
RPT = NPAD // NS
DEG_WIN = 8

RB = 1024
FB = 1000

_mesh = plsc.VectorSubcoreMesh(core_axis_name="c", subcore_axis_name="s")


@functools.partial(
    pl.kernel,
    out_type=jax.ShapeDtypeStruct((NC, NPAD), jnp.float32),
    mesh=_mesh,
    scratch_types=[
        pltpu.VMEM((NBATCH, BB), jnp.int32),
        pltpu.VMEM((BB,), jnp.float32),
        pltpu.VMEM((RPT,), jnp.float32),
        pltpu.VMEM_SHARED((NPAD,), jnp.float32),
        pltpu.SemaphoreType.DMA,
    ],
)
def _deg_kernel(dst_hbm, deg_out, idv, ones_v, zb_v, sdeg, sem_s):
    cid = lax.axis_index("c")
    sid = lax.axis_index("s")
    wid = sid * NC + cid

    pltpu.sync_copy(dst_hbm.at[wid], idv)

    def fill(i, _):
        ones_v[pl.ds(i * 16, 16)] = jnp.full((16,), 1.0, jnp.float32)
        return 0
    lax.fori_loop(0, BB // 16, fill, 0)

    def fill0(i, _):
        zb_v[pl.ds(i * 16, 16)] = jnp.zeros((16,), jnp.float32)
        return 0
    lax.fori_loop(0, RPT // 16, fill0, 0)

    pltpu.sync_copy(zb_v, sdeg.at[pl.ds(sid * RPT, RPT)])
    plsc.subcore_barrier()

    def swait(_i, _):
        pltpu.make_async_copy(ones_v, sdeg.at[idv.at[0]], sem_s).wait()
        return 0

    def step(i, _):
        pltpu.async_copy(ones_v, sdeg.at[idv.at[i]], sem_s, add=True)

        @pl.when(i >= DEG_WIN)
        def _():
            swait(0, 0)
        return 0
    lax.fori_loop(0, NBATCH, step, 0)
    lax.fori_loop(0, DEG_WIN, swait, 0)

    plsc.subcore_barrier()
    pltpu.sync_copy(sdeg.at[pl.ds(sid * RPT, RPT)],
                    deg_out.at[cid, pl.ds(sid * RPT, RPT)])


@functools.partial(
    pl.kernel,
    out_type=jax.ShapeDtypeStruct((NC, ACC, D), jnp.float32),
    mesh=_mesh,
    scratch_types=[
        pltpu.VMEM((PBB,), jnp.int32),
        pltpu.VMEM((PBB,), jnp.int32),
        pltpu.VMEM((PBB,), jnp.int32),
        pltpu.VMEM((PBB,), jnp.int32),
        pltpu.VMEM((PBB, D), jnp.float32),
        pltpu.VMEM((PBB, D), jnp.float32),
        pltpu.VMEM_SHARED((ACC, D), jnp.float32),
        pltpu.SemaphoreType.DMA,
        pltpu.SemaphoreType.DMA,
        pltpu.SemaphoreType.DMA,
        pltpu.SemaphoreType.DMA,
        pltpu.SemaphoreType.DMA,
        pltpu.SemaphoreType.DMA,
        pltpu.SemaphoreType.DMA,
    ],
)
def _prop_kernel(g_hbm, src_hbm, dst_hbm, parts_out,
                 is0, is1, id0, id1, r0, r1, sacc,
                 sem_g, sem_s0, sem_s1, sem_is0, sem_is1, sem_id0, sem_id1):
    cid = lax.axis_index("c")
    sid = lax.axis_index("s")
    wid = sid * NC + cid

    def is_start(i, buf, sem):
        pltpu.async_copy(src_hbm.at[wid, i], buf, sem)

    def id_start(i, buf, sem):
        pltpu.async_copy(dst_hbm.at[wid, i], buf, sem)

    def idx_wait(sem):
        pltpu.make_async_copy(src_hbm.at[wid, 0], is0, sem).wait()

    def gather_start(buf_i, buf_r):
        pltpu.async_copy(g_hbm.at[buf_i], buf_r, sem_g)

    def gather_wait():
        pltpu.make_async_copy(g_hbm.at[is0], r0, sem_g).wait()

    def scatter_start(buf_i, buf_r, sem):
        pltpu.async_copy(buf_r, sacc.at[buf_i], sem, add=True)

    def scatter_wait(sem):
        pltpu.make_async_copy(r0, sacc.at[id0], sem).wait()

    is_start(0, is0, sem_is0)
    id_start(0, id0, sem_id0)

    def fill0(i, _):
        r1[i // 8, pl.ds((i % 8) * 16, 16)] = jnp.zeros((16,), jnp.float32)
        return 0
    lax.fori_loop(0, PBB * (D // 16), fill0, 0)
    zbase = sid * ZPT
    for off, nrows in ((0, 128), (128, 128), (256, 128), (384, 128),
                       (512, 120)):
        pltpu.sync_copy(r1.at[pl.ds(0, nrows)],
                        sacc.at[pl.ds(zbase + off, nrows)])
    plsc.subcore_barrier()

    idx_wait(sem_is0)
    gather_start(is0, r0)
    is_start(1, is1, sem_is1)

    def pair(k, _):
        i0 = 2 * k
        gather_wait()
        is_start(i0 + 2, is0, sem_is0)
        idx_wait(sem_id0)
        scatter_start(id0, r0, sem_s0)
        idx_wait(sem_is1)

        @pl.when(k > 0)
        def _():
            scatter_wait(sem_s1)
        id_start(i0 + 1, id1, sem_id1)
        gather_start(is1, r1)
        gather_wait()

        @pl.when(i0 + 3 < PNB)
        def _():
            is_start(i0 + 3, is1, sem_is1)
        idx_wait(sem_id1)
        scatter_start(id1, r1, sem_s1)
        idx_wait(sem_is0)
        scatter_wait(sem_s0)
        id_start(i0 + 2, id0, sem_id0)
        gather_start(is0, r0)
        return 0
    lax.fori_loop(0, PNB // 2, pair, 0)

    gather_wait()
    idx_wait(sem_id0)
    scatter_start(id0, r0, sem_s0)
    scatter_wait(sem_s1)
    scatter_wait(sem_s0)

    plsc.subcore_barrier()
    pltpu.sync_copy(sacc.at[pl.ds(sid * ZPT, ZPT)],
                    parts_out.at[cid, pl.ds(sid * ZPT, ZPT)])


def _transform(x, W, degs3):
    def body(x_ref, w_ref, deg_ref, g_ref):
        deg = deg_ref[0] + deg_ref[1] + 1.0
        dis = lax.rsqrt(deg)
        h = jnp.dot(x_ref[...], w_ref[...], preferred_element_type=jnp.float32)
        g_ref[...] = h * dis

    return pl.pallas_call(
        body,
        grid=(N // FB,),
        in_specs=[
            pl.BlockSpec((FB, D), lambda i: (i, 0)),
            pl.BlockSpec((D, D), lambda i: (0, 0)),
            pl.BlockSpec((NC, FB, 1), lambda i: (0, i, 0)),
        ],
        out_specs=pl.BlockSpec((FB, D), lambda i: (i, 0)),
        out_shape=jax.ShapeDtypeStruct((N, D), jnp.float32),
    )(x, W, degs3)


def _finalize(parts, g, degs3, b):
    def body(p_ref, g_ref, deg_ref, b_ref, o_ref):
        deg = deg_ref[0] + deg_ref[1] + 1.0
        dis = lax.rsqrt(deg)
        s = p_ref[0] + p_ref[1] + g_ref[...]
        o_ref[...] = jnp.tanh(s * dis + b_ref[...][None, :])

    return pl.pallas_call(
        body,
        grid=(N // FB,),
        in_specs=[
            pl.BlockSpec((NC, FB, D), lambda i: (0, i, 0)),
            pl.BlockSpec((FB, D), lambda i: (i, 0)),
            pl.BlockSpec((NC, FB, 1), lambda i: (0, i, 0)),
            pl.BlockSpec((D,), lambda i: (0,)),
        ],
        out_specs=pl.BlockSpec((FB, D), lambda i: (i, 0)),
        out_shape=jax.ShapeDtypeStruct((N, D), jnp.float32),
    )(parts, g, degs3, b)


def kernel(x, edge_index, W, b):
    src = edge_index[0].astype(jnp.int32)
    dst = edge_index[1].astype(jnp.int32)
    npad = EPAD - E
    pad_src = jnp.arange(npad, dtype=jnp.int32) % N
    pad_dst = N + (jnp.arange(npad, dtype=jnp.int32) % (ACC - N))
    srcp = jnp.concatenate([src, pad_src]).reshape(NW, PNB, PBB)
    dstp = jnp.concatenate([dst, pad_dst]).reshape(NW, PNB, PBB)
    dstd = dst.reshape(NW, NBATCH, BB)

    degs = _deg_kernel(dstd)
    degs3 = degs[:, :N, None]
    g = _transform(x, W, degs3)
    parts = _prop_kernel(g, srcp, dstp)
    return _finalize(parts, g, degs3, b)


# speedup vs baseline: 1.0247x; 1.0247x over previous
import functools

import jax
import jax.numpy as jnp
from jax import lax
from jax.experimental import pallas as pl
from jax.experimental.pallas import tpu as pltpu
from jax.experimental.pallas import tpu_sc as plsc

N = 10000
E = 320000
D = 128
NPAD = 10240
NC = 2
NS = 16
NW = NC * NS

PBB = 128
PNB = 79
EPWP = PNB * PBB
EPAD = NW * EPWP
ACC = 10240
ZPT = ACC // NS

BB = 80
NBATCH = (E // NW) // BB
RPT = NPAD // NS
DEG_WIN = 8

RB = 1024
FB = 1000

_mesh = plsc.VectorSubcoreMesh(core_axis_name="c", subcore_axis_name="s")


@functools.partial(
    pl.kernel,
    out_type=jax.ShapeDtypeStruct((NC, NPAD), jnp.float32),
    mesh=_mesh,
    scratch_types=[
        pltpu.VMEM((NBATCH, BB), jnp.int32),
        pltpu.VMEM((BB,), jnp.float32),
        pltpu.VMEM((RPT,), jnp.float32),
        pltpu.VMEM_SHARED((NPAD,), jnp.float32),
        pltpu.SemaphoreType.DMA,
    ],
)
def _deg_kernel(dst_hbm, deg_out, idv, ones_v, zb_v, sdeg, sem_s):
    cid = lax.axis_index("c")
    sid = lax.axis_index("s")
    wid = sid * NC + cid

    pltpu.sync_copy(dst_hbm.at[wid], idv)

    def fill(i, _):
        ones_v[pl.ds(i * 16, 16)] = jnp.full((16,), 1.0, jnp.float32)
        return 0
    lax.fori_loop(0, BB // 16, fill, 0)

    def fill0(i, _):
        zb_v[pl.ds(i * 16, 16)] = jnp.zeros((16,), jnp.float32)
        return 0
    lax.fori_loop(0, RPT // 16, fill0, 0)

    pltpu.sync_copy(zb_v, sdeg.at[pl.ds(sid * RPT, RPT)])
    plsc.subcore_barrier()

    def swait(_i, _):
        pltpu.make_async_copy(ones_v, sdeg.at[idv.at[0]], sem_s).wait()
        return 0

    def step(i, _):
        pltpu.async_copy(ones_v, sdeg.at[idv.at[i]], sem_s, add=True)

        @pl.when(i >= DEG_WIN)
        def _():
            swait(0, 0)
        return 0
    lax.fori_loop(0, NBATCH, step, 0)
    lax.fori_loop(0, DEG_WIN, swait, 0)

    plsc.subcore_barrier()
    pltpu.sync_copy(sdeg.at[pl.ds(sid * RPT, RPT)],
                    deg_out.at[cid, pl.ds(sid * RPT, RPT)])


@functools.partial(
    pl.kernel,
    out_type=jax.ShapeDtypeStruct((NC, ACC, D), jnp.float32),
    mesh=_mesh,
    scratch_types=[
        pltpu.VMEM((PBB,), jnp.int32),
        pltpu.VMEM((PBB,), jnp.int32),
        pltpu.VMEM((PBB,), jnp.int32),
        pltpu.VMEM((PBB,), jnp.int32),
        pltpu.VMEM((PBB, D), jnp.float32),
        pltpu.VMEM((PBB, D), jnp.float32),
        pltpu.VMEM_SHARED((ACC, D), jnp.float32),
        pltpu.SemaphoreType.DMA,
        pltpu.SemaphoreType.DMA,
        pltpu.SemaphoreType.DMA,
        pltpu.SemaphoreType.DMA,
        pltpu.SemaphoreType.DMA,
        pltpu.SemaphoreType.DMA,
        pltpu.SemaphoreType.DMA,
    ],
)
def _prop_kernel(g_hbm, src_hbm, dst_hbm, parts_out,
                 is0, is1, id0, id1, r0, r1, sacc,
                 sem_g, sem_s0, sem_s1, sem_is0, sem_is1, sem_id0, sem_id1):
    cid = lax.axis_index("c")
    sid = lax.axis_index("s")
    wid = sid * NC + cid

    def is_start(i, buf, sem):
        pltpu.async_copy(src_hbm.at[wid, i], buf, sem)

    def id_start(i, buf, sem):
        pltpu.async_copy(dst_hbm.at[wid, i], buf, sem)

    def idx_wait(sem):
        pltpu.make_async_copy(src_hbm.at[wid, 0], is0, sem).wait()

    def gather_start(buf_i, buf_r):
        pltpu.async_copy(g_hbm.at[buf_i], buf_r, sem_g)

    def gather_wait():
        pltpu.make_async_copy(g_hbm.at[is0], r0, sem_g).wait()

    def scatter_start(buf_i, buf_r, sem):
        pltpu.async_copy(buf_r, sacc.at[buf_i], sem, add=True)

    def scatter_wait(sem):
        pltpu.make_async_copy(r0, sacc.at[id0], sem).wait()

    is_start(0, is0, sem_is0)
    id_start(0, id0, sem_id0)

    def fill0(i, _):
        r1[i // 8, pl.ds((i % 8) * 16, 16)] = jnp.zeros((16,), jnp.float32)
        return 0
    lax.fori_loop(0, PBB * (D // 16), fill0, 0)
    zbase = sid * ZPT
    for off in (0, 128, 256, 384, 512):
        pltpu.sync_copy(r1, sacc.at[pl.ds(zbase + off, PBB)])
    plsc.subcore_barrier()

    idx_wait(sem_is0)
    gather_start(is0, r0)
    is_start(1, is1, sem_is1)

    def pair(k, _):
        i0 = 2 * k
        gather_wait()
        is_start(i0 + 2, is0, sem_is0)
        idx_wait(sem_id0)
        scatter_start(id0, r0, sem_s0)
        idx_wait(sem_is1)

        @pl.when(k > 0)
        def _():
            scatter_wait(sem_s1)
        id_start(i0 + 1, id1, sem_id1)
        gather_start(is1, r1)
        gather_wait()

        @pl.when(i0 + 3 < PNB)
        def _():
            is_start(i0 + 3, is1, sem_is1)
        idx_wait(sem_id1)
        scatter_start(id1, r1, sem_s1)
        idx_wait(sem_is0)
        scatter_wait(sem_s0)
        id_start(i0 + 2, id0, sem_id0)
        gather_start(is0, r0)
        return 0
    lax.fori_loop(0, PNB // 2, pair, 0)

    gather_wait()
    idx_wait(sem_id0)
    scatter_start(id0, r0, sem_s0)
    scatter_wait(sem_s1)
    scatter_wait(sem_s0)

    plsc.subcore_barrier()
    pltpu.sync_copy(sacc.at[pl.ds(sid * ZPT, ZPT)],
                    parts_out.at[cid, pl.ds(sid * ZPT, ZPT)])


def _transform(x_pad, W, degs):
    def body(x_ref, w_ref, deg_ref, g_ref):
        deg = deg_ref[0, :] + deg_ref[1, :] + 1.0
        dis = lax.rsqrt(deg)
        h = jnp.dot(x_ref[...], w_ref[...], preferred_element_type=jnp.float32)
        g_ref[...] = h * dis[:, None]

    return pl.pallas_call(
        body,
        grid=(NPAD // RB,),
        in_specs=[
            pl.BlockSpec((RB, D), lambda i: (i, 0)),
            pl.BlockSpec((D, D), lambda i: (0, 0)),
            pl.BlockSpec((NC, RB), lambda i: (0, i)),
        ],
        out_specs=pl.BlockSpec((RB, D), lambda i: (i, 0)),
        out_shape=jax.ShapeDtypeStruct((NPAD, D), jnp.float32),
    )(x_pad, W, degs)


def _finalize(parts, g, degs, b):
    def body(p_ref, g_ref, deg_ref, b_ref, o_ref):
        deg = deg_ref[0, :] + deg_ref[1, :] + 1.0
        dis = lax.rsqrt(deg)
        s = p_ref[0] + p_ref[1] + g_ref[...]
        o_ref[...] = jnp.tanh(s * dis[:, None] + b_ref[...][None, :])

    return pl.pallas_call(
        body,
        grid=(NPAD // RB,),
        in_specs=[
            pl.BlockSpec((NC, RB, D), lambda i: (0, i, 0)),
            pl.BlockSpec((RB, D), lambda i: (i, 0)),
            pl.BlockSpec((NC, RB), lambda i: (0, i)),
            pl.BlockSpec((D,), lambda i: (0,)),
        ],
        out_specs=pl.BlockSpec((RB, D), lambda i: (i, 0)),
        out_shape=jax.ShapeDtypeStruct((NPAD, D), jnp.float32),
    )(parts, g, degs, b)


def kernel(x, edge_index, W, b):
    src = edge_index[0].astype(jnp.int32)
    dst = edge_index[1].astype(jnp.int32)
    npad = EPAD - E
    pad_src = jnp.arange(npad, dtype=jnp.int32) % N
    pad_dst = N + (jnp.arange(npad, dtype=jnp.int32) % (ACC - N))
    srcp = jnp.concatenate([src, pad_src]).reshape(NW, PNB, PBB)
    dstp = jnp.concatenate([dst, pad_dst]).reshape(NW, PNB, PBB)
    dstd = dst.reshape(NW, NBATCH, BB)
    x_pad = jnp.pad(x, ((0, NPAD - N), (0, 0)))

    degs = _deg_kernel(dstd)
    g = _transform(x_pad, W, degs)
    parts = _prop_kernel(g, srcp, dstp)
    return _finalize(parts, g, degs, b)[:N]


# speedup vs baseline: 1.0505x; 1.0252x over previous
import functools

import jax
import jax.numpy as jnp
from jax import lax
from jax.experimental import pallas as pl
from jax.experimental.pallas import tpu as pltpu
from jax.experimental.pallas import tpu_sc as plsc

N = 10000
E = 320000
D = 128
NPAD = 10240
NC = 2
NS = 16
NW = NC * NS

PBB = 128
PNB = 79
EPWP = PNB * PBB
EPAD = NW * EPWP
ACC = 10240
ZPT = ACC // NS

BB = 80
NBATCH = (E // NW) // BB
RPT = NPAD // NS
DEG_WIN = 8

RB = 1024
FB = 1000

_mesh = plsc.VectorSubcoreMesh(core_axis_name="c", subcore_axis_name="s")


@functools.partial(
    pl.kernel,
    out_type=jax.ShapeDtypeStruct((NC, NPAD), jnp.float32),
    mesh=_mesh,
    scratch_types=[
        pltpu.VMEM((NBATCH, BB), jnp.int32),
        pltpu.VMEM((BB,), jnp.float32),
        pltpu.VMEM((RPT,), jnp.float32),
        pltpu.VMEM_SHARED((NPAD,), jnp.float32),
        pltpu.SemaphoreType.DMA,
    ],
)
def _deg_kernel(dst_hbm, deg_out, idv, ones_v, zb_v, sdeg, sem_s):
    cid = lax.axis_index("c")
    sid = lax.axis_index("s")
    wid = sid * NC + cid

    pltpu.sync_copy(dst_hbm.at[wid], idv)

    def fill(i, _):
        ones_v[pl.ds(i * 16, 16)] = jnp.full((16,), 1.0, jnp.float32)
        return 0
    lax.fori_loop(0, BB // 16, fill, 0)

    def fill0(i, _):
        zb_v[pl.ds(i * 16, 16)] = jnp.zeros((16,), jnp.float32)
        return 0
    lax.fori_loop(0, RPT // 16, fill0, 0)

    pltpu.sync_copy(zb_v, sdeg.at[pl.ds(sid * RPT, RPT)])
    plsc.subcore_barrier()

    def swait(_i, _):
        pltpu.make_async_copy(ones_v, sdeg.at[idv.at[0]], sem_s).wait()
        return 0

    def step(i, _):
        pltpu.async_copy(ones_v, sdeg.at[idv.at[i]], sem_s, add=True)

        @pl.when(i >= DEG_WIN)
        def _():
            swait(0, 0)
        return 0
    lax.fori_loop(0, NBATCH, step, 0)
    lax.fori_loop(0, DEG_WIN, swait, 0)

    plsc.subcore_barrier()
    pltpu.sync_copy(sdeg.at[pl.ds(sid * RPT, RPT)],
                    deg_out.at[cid, pl.ds(sid * RPT, RPT)])


@functools.partial(
    pl.kernel,
    out_type=jax.ShapeDtypeStruct((NC, ACC, D), jnp.float32),
    mesh=_mesh,
    scratch_types=[
        pltpu.VMEM((PBB,), jnp.int32),
        pltpu.VMEM((PBB,), jnp.int32),
        pltpu.VMEM((PBB,), jnp.int32),
        pltpu.VMEM((PBB,), jnp.int32),
        pltpu.VMEM((PBB, D), jnp.float32),
        pltpu.VMEM((PBB, D), jnp.float32),
        pltpu.VMEM_SHARED((ACC, D), jnp.float32),
        pltpu.SemaphoreType.DMA,
        pltpu.SemaphoreType.DMA,
        pltpu.SemaphoreType.DMA,
        pltpu.SemaphoreType.DMA,
        pltpu.SemaphoreType.DMA,
        pltpu.SemaphoreType.DMA,
        pltpu.SemaphoreType.DMA,
    ],
)
def _prop_kernel(g_hbm, src_hbm, dst_hbm, parts_out,
                 is0, is1, id0, id1, r0, r1, sacc,
                 sem_g, sem_s0, sem_s1, sem_is0, sem_is1, sem_id0, sem_id1):
    cid = lax.axis_index("c")
    sid = lax.axis_index("s")
    wid = sid * NC + cid

    def is_start(i, buf, sem):
        pltpu.async_copy(src_hbm.at[wid, i], buf, sem)

    def id_start(i, buf, sem):
        pltpu.async_copy(dst_hbm.at[wid, i], buf, sem)

    def idx_wait(sem):
        pltpu.make_async_copy(src_hbm.at[wid, 0], is0, sem).wait()

    def gather_start(buf_i, buf_r):
        pltpu.async_copy(g_hbm.at[buf_i], buf_r, sem_g)

    def gather_wait():
        pltpu.make_async_copy(g_hbm.at[is0], r0, sem_g).wait()

    def scatter_start(buf_i, buf_r, sem):
        pltpu.async_copy(buf_r, sacc.at[buf_i], sem, add=True)

    def scatter_wait(sem):
        pltpu.make_async_copy(r0, sacc.at[id0], sem).wait()

    is_start(0, is0, sem_is0)
    id_start(0, id0, sem_id0)

    def fill0(i, _):
        r1[i // 8, pl.ds((i % 8) * 16, 16)] = jnp.zeros((16,), jnp.float32)
        return 0
    lax.fori_loop(0, PBB * (D // 16), fill0, 0)
    zbase = sid * ZPT
    for off in (0, 128, 256, 384, 512):
        pltpu.sync_copy(r1, sacc.at[pl.ds(zbase + off, PBB)])
    plsc.subcore_barrier()

    idx_wait(sem_is0)
    gather_start(is0, r0)
    is_start(1, is1, sem_is1)

    def pair(k, _):
        i0 = 2 * k
        gather_wait()
        is_start(i0 + 2, is0, sem_is0)
        idx_wait(sem_id0)
        scatter_start(id0, r0, sem_s0)
        idx_wait(sem_is1)

        @pl.when(k > 0)
        def _():
            scatter_wait(sem_s1)
        id_start(i0 + 1, id1, sem_id1)
        gather_start(is1, r1)
        gather_wait()

        @pl.when(i0 + 3 < PNB)
        def _():
            is_start(i0 + 3, is1, sem_is1)
        idx_wait(sem_id1)
        scatter_start(id1, r1, sem_s1)
        idx_wait(sem_is0)
        scatter_wait(sem_s0)
        id_start(i0 + 2, id0, sem_id0)
        gather_start(is0, r0)
        return 0
    lax.fori_loop(0, PNB // 2, pair, 0)

    gather_wait()
    idx_wait(sem_id0)
    scatter_start(id0, r0, sem_s0)
    scatter_wait(sem_s1)
    scatter_wait(sem_s0)

    plsc.subcore_barrier()
    pltpu.sync_copy(sacc.at[pl.ds(sid * ZPT, ZPT)],
                    parts_out.at[cid, pl.ds(sid * ZPT, ZPT)])


def _transform(x_pad, W, degs):
    def body(x_ref, w_ref, deg_ref, g_ref):
        deg = deg_ref[0, :] + deg_ref[1, :] + 1.0
        dis = lax.rsqrt(deg)
        h = jnp.dot(x_ref[...], w_ref[...], preferred_element_type=jnp.float32)
        g_ref[...] = h * dis[:, None]

    return pl.pallas_call(
        body,
        grid=(NPAD // RB,),
        in_specs=[
            pl.BlockSpec((RB, D), lambda i: (i, 0)),
            pl.BlockSpec((D, D), lambda i: (0, 0)),
            pl.BlockSpec((NC, RB), lambda i: (0, i)),
        ],
        out_specs=pl.BlockSpec((RB, D), lambda i: (i, 0)),
        out_shape=jax.ShapeDtypeStruct((NPAD, D), jnp.float32),
    )(x_pad, W, degs)


def _finalize(parts, g, degs3, b):
    def body(p_ref, g_ref, deg_ref, b_ref, o_ref):
        deg = deg_ref[0] + deg_ref[1] + 1.0
        dis = lax.rsqrt(deg)
        s = p_ref[0] + p_ref[1] + g_ref[...]
        o_ref[...] = jnp.tanh(s * dis + b_ref[...][None, :])

    return pl.pallas_call(
        body,
        grid=(N // FB,),
        in_specs=[
            pl.BlockSpec((NC, FB, D), lambda i: (0, i, 0)),
            pl.BlockSpec((FB, D), lambda i: (i, 0)),
            pl.BlockSpec((NC, FB, 1), lambda i: (0, i, 0)),
            pl.BlockSpec((D,), lambda i: (0,)),
        ],
        out_specs=pl.BlockSpec((FB, D), lambda i: (i, 0)),
        out_shape=jax.ShapeDtypeStruct((N, D), jnp.float32),
    )(parts, g, degs3, b)


def kernel(x, edge_index, W, b):
    src = edge_index[0].astype(jnp.int32)
    dst = edge_index[1].astype(jnp.int32)
    npad = EPAD - E
    pad_src = jnp.arange(npad, dtype=jnp.int32) % N
    pad_dst = N + (jnp.arange(npad, dtype=jnp.int32) % (ACC - N))
    srcp = jnp.concatenate([src, pad_src]).reshape(NW, PNB, PBB)
    dstp = jnp.concatenate([dst, pad_dst]).reshape(NW, PNB, PBB)
    dstd = dst.reshape(NW, NBATCH, BB)
    x_pad = jnp.pad(x, ((0, NPAD - N), (0, 0)))

    degs = _deg_kernel(dstd)
    g = _transform(x_pad, W, degs)
    parts = _prop_kernel(g, srcp, dstp)
    return _finalize(parts, g, degs[:, :N, None], b)
